# Initial kernel scaffold; baseline (speedup 1.0000x reference)
import jax
import jax.numpy as jnp
from jax.experimental import pallas as pl


def kernel(x, edge_index, edge_weight, W1, W2):
    raise NotImplementedError("write your pallas kernel here")


# speedup vs baseline: 17.5687x; 17.5687x over previous
import functools

import jax
import jax.numpy as jnp
from jax import lax
from jax.experimental import pallas as pl
from jax.experimental.pallas import tpu as pltpu
from jax.experimental.pallas import tpu_sc as plsc

_NC = 2
_NS = 16
_NW = _NC * _NS

_SUB = 125


def _matmul_tc(x, w):
    def body(x_ref, w_ref, o_ref):
        o_ref[...] = jnp.dot(x_ref[...], w_ref[...],
                             preferred_element_type=jnp.float32)
    return pl.pallas_call(
        body,
        out_shape=jax.ShapeDtypeStruct((x.shape[0], w.shape[1]), jnp.float32),
    )(x, w)


def _add_relu_tc(p):
    def body(p_ref, o_ref):
        o_ref[...] = jnp.maximum(p_ref[0] + p_ref[1], 0.0)
    return pl.pallas_call(
        body,
        out_shape=jax.ShapeDtypeStruct(p.shape[1:], jnp.float32),
    )(p)


def _add_matmul_tc(q, w):
    def body(q_ref, w_ref, o_ref):
        o_ref[...] = jnp.dot(q_ref[0] + q_ref[1], w_ref[...],
                             preferred_element_type=jnp.float32)
    return pl.pallas_call(
        body,
        out_shape=jax.ShapeDtypeStruct((q.shape[1], w.shape[1]), jnp.float32),
    )(q, w)


def _make_aggregate(N, E, D):
    epw = E // _NW
    B = 2000
    nblk = epw // B
    nsub = B // _SUB
    zrows = (N // _NS) // 8 * 8
    tail = N - zrows * _NS
    assert epw * _NW == E and nblk * B == epw and nsub * _SUB == B
    assert 0 <= tail <= B and zrows <= B and tail % 8 == 0

    mesh = plsc.VectorSubcoreMesh(core_axis_name="c", subcore_axis_name="s")

    @functools.partial(
        pl.kernel,
        out_type=jax.ShapeDtypeStruct((_NC, N, D), jnp.float32),
        mesh=mesh,
        scratch_types=[
            pltpu.VMEM((nsub, _SUB), jnp.int32),
            pltpu.VMEM((nsub, _SUB), jnp.int32),
            pltpu.VMEM((B,), jnp.float32),
            pltpu.VMEM((B, D), jnp.float32),
            pltpu.VMEM_SHARED((N, D), jnp.float32),
            pltpu.SemaphoreType.DMA,
        ],
        compiler_params=pltpu.CompilerParams(use_tc_tiling_on_sc=False),
    )
    def agg(table, col2, row2, w_hbm, out, colv, rowv, wv, rowsv, acc, sem):
        cid = lax.axis_index("c")
        sid = lax.axis_index("s")
        wid = sid * _NC + cid

        def zbody(j, c):
            rowsv[j, :] = jnp.zeros((D,), jnp.float32)
            return c
        lax.fori_loop(0, max(zrows, tail), zbody, 0)
        pltpu.sync_copy(rowsv.at[pl.ds(0, zrows)],
                        acc.at[pl.ds(sid * zrows, zrows)])
        if tail:
            @pl.when(sid == 0)
            def _():
                pltpu.sync_copy(rowsv.at[pl.ds(0, tail)],
                                acc.at[pl.ds(_NS * zrows, tail)])
        plsc.subcore_barrier()

        def blk(b, c):
            off = pl.multiple_of(wid * epw + b * B, B)
            r0 = pl.multiple_of(off // _SUB, nsub)
            pltpu.sync_copy(col2.at[pl.ds(r0, nsub)], colv)
            pltpu.sync_copy(row2.at[pl.ds(r0, nsub)], rowv)
            pltpu.sync_copy(w_hbm.at[pl.ds(off, B)], wv)
            descs = [
                pltpu.async_copy(table.at[colv.at[j]],
                                 rowsv.at[pl.ds(j * _SUB, _SUB)], sem)
                for j in range(nsub)
            ]
            for d in descs:
                d.wait()
            def sbody(g, cc):
                w16 = wv[pl.ds(pl.multiple_of(g * 16, 16), 16)]
                for l in range(16):
                    j = g * 16 + l
                    rowsv[j, :] = rowsv[j, :] * w16[l]
                return cc
            lax.fori_loop(0, B // 16, sbody, 0)
            sdescs = [
                pltpu.async_copy(rowsv.at[pl.ds(j * _SUB, _SUB)],
                                 acc.at[rowv.at[j]], sem, add=True)
                for j in range(nsub)
            ]
            for d in sdescs:
                d.wait()
            return c
        lax.fori_loop(0, nblk, blk, 0)

        plsc.subcore_barrier()
        pltpu.sync_copy(acc.at[pl.ds(sid * zrows, zrows)],
                        out.at[cid].at[pl.ds(sid * zrows, zrows)])
        if tail:
            @pl.when(sid == 0)
            def _():
                pltpu.sync_copy(acc.at[pl.ds(_NS * zrows, tail)],
                                out.at[cid].at[pl.ds(_NS * zrows, tail)])

    return agg


def kernel(x, edge_index, edge_weight, W1, W2):
    N = x.shape[0]
    E = edge_index.shape[1]
    D = W1.shape[1]
    row2 = edge_index[0].reshape(E // _SUB, _SUB)
    col2 = edge_index[1].reshape(E // _SUB, _SUB)

    agg = _make_aggregate(N, E, D)

    h1 = _matmul_tc(x, W1)
    p = agg(h1, col2, row2, edge_weight)
    h2 = _add_relu_tc(p)
    q = agg(h2, col2, row2, edge_weight)
    return _add_matmul_tc(q, W2)


# speedup vs baseline: 20.2708x; 1.1538x over previous
import functools

import jax
import jax.numpy as jnp
from jax import lax
from jax.experimental import pallas as pl
from jax.experimental.pallas import tpu as pltpu
from jax.experimental.pallas import tpu_sc as plsc

_NC = 2
_NS = 16
_NW = _NC * _NS

_SUB = 125


def _matmul_tc(x, w):
    def body(x_ref, w_ref, o_ref):
        o_ref[...] = jnp.dot(x_ref[...], w_ref[...],
                             preferred_element_type=jnp.float32)
    return pl.pallas_call(
        body,
        out_shape=jax.ShapeDtypeStruct((x.shape[0], w.shape[1]), jnp.float32),
    )(x, w)


def _add_relu_tc(p):
    def body(p_ref, o_ref):
        o_ref[...] = jnp.maximum(p_ref[0] + p_ref[1], 0.0)
    return pl.pallas_call(
        body,
        out_shape=jax.ShapeDtypeStruct(p.shape[1:], jnp.float32),
    )(p)


def _add_matmul_tc(q, w):
    def body(q_ref, w_ref, o_ref):
        o_ref[...] = jnp.dot(q_ref[0] + q_ref[1], w_ref[...],
                             preferred_element_type=jnp.float32)
    return pl.pallas_call(
        body,
        out_shape=jax.ShapeDtypeStruct((q.shape[1], w.shape[1]), jnp.float32),
    )(q, w)


def _make_aggregate(N, E, D):
    epw = E // _NW
    B = 2000
    nblk = epw // B
    nsub = B // _SUB
    zrows = (N // _NS) // 8 * 8
    tail = N - zrows * _NS
    assert epw * _NW == E and nblk * B == epw and nsub * _SUB == B
    assert 0 <= tail <= B and zrows <= B and tail % 8 == 0

    mesh = plsc.VectorSubcoreMesh(core_axis_name="c", subcore_axis_name="s")

    @functools.partial(
        pl.kernel,
        out_type=jax.ShapeDtypeStruct((_NC, N, D), jnp.float32),
        mesh=mesh,
        scratch_types=[
            pltpu.VMEM((2, nsub, _SUB), jnp.int32),
            pltpu.VMEM((2, nsub, _SUB), jnp.int32),
            pltpu.VMEM((2, B), jnp.float32),
            pltpu.VMEM((2, B, D), jnp.float32),
            pltpu.VMEM_SHARED((N, D), jnp.float32),
            pltpu.SemaphoreType.DMA,
            pltpu.SemaphoreType.DMA,
            pltpu.SemaphoreType.DMA,
            pltpu.SemaphoreType.DMA,
        ],
        compiler_params=pltpu.CompilerParams(use_tc_tiling_on_sc=False),
    )
    def agg(table, col2, row2, w_hbm, out, colv, rowv, wv, rowsv, acc,
            gsem0, gsem1, ssem0, ssem1):
        cid = lax.axis_index("c")
        sid = lax.axis_index("s")
        wid = sid * _NC + cid
        gsems = (gsem0, gsem1)
        ssems = (ssem0, ssem1)

        def zbody(j, c):
            rowsv[0, j, :] = jnp.zeros((D,), jnp.float32)
            return c
        lax.fori_loop(0, max(zrows, tail), zbody, 0)
        pltpu.sync_copy(rowsv.at[0, pl.ds(0, zrows)],
                        acc.at[pl.ds(sid * zrows, zrows)])
        if tail:
            @pl.when(sid == 0)
            def _():
                pltpu.sync_copy(rowsv.at[0, pl.ds(0, tail)],
                                acc.at[pl.ds(_NS * zrows, tail)])
        plsc.subcore_barrier()

        def fire_gather(b):
            s = b % 2
            off = pl.multiple_of(wid * epw + b * B, B)
            r0 = pl.multiple_of(off // _SUB, nsub)
            pltpu.sync_copy(col2.at[pl.ds(r0, nsub)], colv.at[s])
            pltpu.sync_copy(row2.at[pl.ds(r0, nsub)], rowv.at[s])
            pltpu.sync_copy(w_hbm.at[pl.ds(off, B)], wv.at[s])
            return [
                pltpu.async_copy(table.at[colv.at[s].at[j]],
                                 rowsv.at[s, pl.ds(j * _SUB, _SUB)], gsems[s])
                for j in range(nsub)
            ]

        def fire_scatter(b):
            s = b % 2
            return [
                pltpu.async_copy(rowsv.at[s, pl.ds(j * _SUB, _SUB)],
                                 acc.at[rowv.at[s].at[j]], ssems[s], add=True)
                for j in range(nsub)
            ]

        def scale(b):
            s = b % 2
            def sbody(g, cc):
                w16 = wv[s, pl.ds(pl.multiple_of(g * 16, 16), 16)]
                for l in range(16):
                    j = g * 16 + l
                    rowsv[s, j, :] = rowsv[s, j, :] * w16[l]
                return cc
            lax.fori_loop(0, B // 16, sbody, 0)

        gd = fire_gather(0)
        sd_prev = None
        for b in range(nblk):
            if b + 1 < nblk:
                if sd_prev is not None:
                    for d in sd_prev:
                        d.wait()
                sd_prev = None
                gd_next = fire_gather(b + 1)
            else:
                gd_next = None
            for d in gd:
                d.wait()
            scale(b)
            if sd_prev is not None:
                for d in sd_prev:
                    d.wait()
            sd_prev = fire_scatter(b)
            gd = gd_next
        for d in sd_prev:
            d.wait()

        plsc.subcore_barrier()
        pltpu.sync_copy(acc.at[pl.ds(sid * zrows, zrows)],
                        out.at[cid].at[pl.ds(sid * zrows, zrows)])
        if tail:
            @pl.when(sid == 0)
            def _():
                pltpu.sync_copy(acc.at[pl.ds(_NS * zrows, tail)],
                                out.at[cid].at[pl.ds(_NS * zrows, tail)])

    return agg


def kernel(x, edge_index, edge_weight, W1, W2):
    N = x.shape[0]
    E = edge_index.shape[1]
    D = W1.shape[1]
    row2 = edge_index[0].reshape(E // _SUB, _SUB)
    col2 = edge_index[1].reshape(E // _SUB, _SUB)

    agg = _make_aggregate(N, E, D)

    h1 = _matmul_tc(x, W1)
    p = agg(h1, col2, row2, edge_weight)
    h2 = _add_relu_tc(p)
    q = agg(h2, col2, row2, edge_weight)
    return _add_matmul_tc(q, W2)
